# baseline (speedup 1.0000x reference)
import functools

import jax
import jax.numpy as jnp
from jax import lax
from jax.experimental import pallas as pl
from jax.experimental.pallas import tpu as pltpu
from jax.experimental.pallas import tpu_sc as plsc

V = 1000000
B, L, D, H = 4096, 200, 64, 128
DP = D // 2
NC, NS = 2, 16
NW = NC * NS
RPW = B // NW
NCHUNK = 2
CHUNK = L // NCHUNK
BLANE = 32
G = 4
NGRP = RPW // G

VB = 6400
TGRID = 40
VS4 = TGRID * VB
NVBLK = V // VB

_mesh = plsc.VectorSubcoreMesh(
    core_axis_name="c", subcore_axis_name="s", num_cores=NC, num_subcores=NS
)


def _transpose_body(q0_ref, q1_ref, q2_ref, q3_ref, o_ref):
    for q, ref in enumerate((q0_ref, q1_ref, q2_ref, q3_ref)):
        tr = lax.transpose(ref[...], (1, 0))
        u = lax.bitcast_convert_type(tr, jnp.uint32)
        ub = (u + (((u >> 16) & 1) + 0x7FFF)) >> 16
        packed_u = ub[:, 0:DP] | (ub[:, DP:D] << 16)
        o_ref[:, pl.ds(q * DP, DP)] = lax.bitcast_convert_type(
            packed_u, jnp.float32
        )


@functools.partial(
    pl.kernel,
    out_type=jax.ShapeDtypeStruct((B, D), jnp.bfloat16),
    mesh=_mesh,
    scratch_types=[
        pltpu.VMEM((RPW, NCHUNK, CHUNK), jnp.int32),
        pltpu.VMEM((2, G, NCHUNK, CHUNK, DP), jnp.float32),
        pltpu.VMEM((RPW, D), jnp.bfloat16),
        pltpu.SemaphoreType.DMA,
        pltpu.SemaphoreType.DMA,
    ],
    compiler_params=pltpu.CompilerParams(
        use_tc_tiling_on_sc=False, needs_layout_passes=False
    ),
)
def _pool_kernel(x_hbm, table_hbm, out_hbm, idx_v, rows_v, out_v, sem0, sem1):
    wid = lax.axis_index("s") * NC + lax.axis_index("c")
    base = wid * RPW
    sems = (sem0, sem1)
    pltpu.sync_copy(x_hbm.at[pl.ds(base, RPW)], idx_v)

    def start(g, p):
        for u in range(G):
            for j in range(NCHUNK):
                pltpu.async_copy(
                    table_hbm.at[idx_v.at[g * G + u, j]],
                    rows_v.at[p, u, j],
                    sems[p],
                )

    def wait(g, p):
        for u in range(G):
            for j in range(NCHUNK):
                pltpu.make_async_copy(
                    table_hbm.at[idx_v.at[g * G + u, j]],
                    rows_v.at[p, u, j],
                    sems[p],
                ).wait()

    def reduce(g, p):
        for u in range(G):
            def red_body(r, accs):
                res = list(accs)
                for j in range(NCHUNK):
                    for c in range(2):
                        v16 = rows_v[p, u, j, r, pl.ds(c * 16, 16)]
                        res[c] = jnp.maximum(
                            res[c], plsc.bitcast(v16, jnp.bfloat16)
                        )
                return tuple(res)

            init = tuple(
                jnp.full((BLANE,), -jnp.inf, jnp.bfloat16) for _ in range(2)
            )
            accs = lax.fori_loop(0, CHUNK, red_body, init)
            for c in range(2):
                out_v[g * G + u, pl.ds(c * BLANE, BLANE)] = accs[c]

    start(0, 0)
    start(1, 1)

    def grp_body(i, carry):
        for p in range(2):
            g = 2 * i + p
            wait(g, p)
            reduce(g, p)
            start(g + 2, p)
        return carry

    lax.fori_loop(0, NGRP // 2 - 1, grp_body, 0)
    for p in range(2):
        g = NGRP - 2 + p
        wait(g, p)
        reduce(g, p)
    pltpu.sync_copy(out_v, out_hbm.at[pl.ds(base, RPW)])


def _head_body(p_ref, w_ref, b_ref, o_ref):
    pooled = p_ref[...].astype(jnp.float32)
    h = lax.dot_general(
        pooled, w_ref[...], (((1,), (1,)), ((), ())),
        preferred_element_type=jnp.float32,
    )
    h = h + b_ref[...]
    s = jnp.sum(h * h, axis=1, keepdims=True)
    o_ref[...] = h * lax.rsqrt(jnp.maximum(s, 1e-24))


def kernel(x, embed_table, W, b):
    xi = x.astype(jnp.int32)
    xg = 4 * jnp.remainder(xi, VS4) + xi // VS4
    x3 = xg.reshape(B, NCHUNK, CHUNK)
    tbT = embed_table.T

    def _qmap(q):
        return lambda i: (0, jnp.minimum(q * TGRID + i, NVBLK))

    tb_lin = pl.pallas_call(
        _transpose_body,
        grid=(TGRID,),
        in_specs=[pl.BlockSpec((D, VB), _qmap(q)) for q in range(4)],
        out_specs=pl.BlockSpec((VB, 4 * DP), lambda i: (i, 0)),
        out_shape=jax.ShapeDtypeStruct((VS4, 4 * DP), jnp.float32),
    )(tbT, tbT, tbT, tbT)
    pooled = _pool_kernel(x3, tb_lin.reshape(4 * VS4, DP))
    k = jnp.arange(D)
    perm = 16 * (k // 32) + (k % 32) // 2 + 32 * (k % 2)
    out = pl.pallas_call(
        _head_body,
        out_shape=jax.ShapeDtypeStruct((B, H), jnp.float32),
    )(pooled, W[:, perm], b.reshape(1, H))
    return out


# speedup vs baseline: 1.2042x; 1.2042x over previous
import functools

import jax
import jax.numpy as jnp
from jax import lax
from jax.experimental import pallas as pl
from jax.experimental.pallas import tpu as pltpu
from jax.experimental.pallas import tpu_sc as plsc

V = 1000000
B, L, D, H = 4096, 200, 64, 128
DP = D // 2
NC, NS = 2, 16
NW = NC * NS
RPW = B // NW
NCHUNK = 2
CHUNK = L // NCHUNK
BLANE = 32
G = 4
NGRP = RPW // G

VB = 6400
TGRID = 40
VS4 = TGRID * VB
NVBLK = V // VB

_mesh = plsc.VectorSubcoreMesh(
    core_axis_name="c", subcore_axis_name="s", num_cores=NC, num_subcores=NS
)


def _transpose_body(q0_ref, q1_ref, q2_ref, q3_ref, o_ref):
    for q, ref in enumerate((q0_ref, q1_ref, q2_ref, q3_ref)):
        u = lax.bitcast_convert_type(ref[...], jnp.uint32)
        ub = (u + (((u >> 16) & 1) + 0x7FFF)) >> 16
        packed_u = ub[0:DP, :] | (ub[DP:D, :] << 16)
        packed = lax.bitcast_convert_type(packed_u, jnp.float32)
        o_ref[:, pl.ds(q * DP, DP)] = lax.transpose(packed, (1, 0))


@functools.partial(
    pl.kernel,
    out_type=jax.ShapeDtypeStruct((B, D), jnp.bfloat16),
    mesh=_mesh,
    scratch_types=[
        pltpu.VMEM((RPW, NCHUNK, CHUNK), jnp.int32),
        pltpu.VMEM((2, G, NCHUNK, CHUNK, DP), jnp.float32),
        pltpu.VMEM((RPW, D), jnp.bfloat16),
        pltpu.SemaphoreType.DMA,
        pltpu.SemaphoreType.DMA,
    ],
    compiler_params=pltpu.CompilerParams(
        use_tc_tiling_on_sc=False, needs_layout_passes=False
    ),
)
def _pool_kernel(x_hbm, table_hbm, out_hbm, idx_v, rows_v, out_v, sem0, sem1):
    wid = lax.axis_index("s") * NC + lax.axis_index("c")
    base = wid * RPW
    sems = (sem0, sem1)
    pltpu.sync_copy(x_hbm.at[pl.ds(base, RPW)], idx_v)

    def start(g, p):
        for u in range(G):
            for j in range(NCHUNK):
                pltpu.async_copy(
                    table_hbm.at[idx_v.at[g * G + u, j]],
                    rows_v.at[p, u, j],
                    sems[p],
                )

    def wait(g, p):
        for u in range(G):
            for j in range(NCHUNK):
                pltpu.make_async_copy(
                    table_hbm.at[idx_v.at[g * G + u, j]],
                    rows_v.at[p, u, j],
                    sems[p],
                ).wait()

    def reduce(g, p):
        for u in range(G):
            def red_body(r, accs):
                res = list(accs)
                for j in range(NCHUNK):
                    for c in range(2):
                        v16 = rows_v[p, u, j, r, pl.ds(c * 16, 16)]
                        res[c] = jnp.maximum(
                            res[c], plsc.bitcast(v16, jnp.bfloat16)
                        )
                return tuple(res)

            init = tuple(
                jnp.full((BLANE,), -jnp.inf, jnp.bfloat16) for _ in range(2)
            )
            accs = lax.fori_loop(0, CHUNK, red_body, init)
            for c in range(2):
                out_v[g * G + u, pl.ds(c * BLANE, BLANE)] = accs[c]

    start(0, 0)
    start(1, 1)

    def grp_body(i, carry):
        for p in range(2):
            g = 2 * i + p
            wait(g, p)
            reduce(g, p)
            start(g + 2, p)
        return carry

    lax.fori_loop(0, NGRP // 2 - 1, grp_body, 0)
    for p in range(2):
        g = NGRP - 2 + p
        wait(g, p)
        reduce(g, p)
    pltpu.sync_copy(out_v, out_hbm.at[pl.ds(base, RPW)])


def _head_body(p_ref, w_ref, b_ref, o_ref):
    pooled = p_ref[...].astype(jnp.float32)
    h = lax.dot_general(
        pooled, w_ref[...], (((1,), (1,)), ((), ())),
        preferred_element_type=jnp.float32,
    )
    h = h + b_ref[...]
    s = jnp.sum(h * h, axis=1, keepdims=True)
    o_ref[...] = h * lax.rsqrt(jnp.maximum(s, 1e-24))


def kernel(x, embed_table, W, b):
    xi = x.astype(jnp.int32)
    xg = 4 * jnp.remainder(xi, VS4) + xi // VS4
    x3 = xg.reshape(B, NCHUNK, CHUNK)
    tbT = embed_table.T

    def _qmap(q):
        return lambda i: (0, jnp.minimum(q * TGRID + i, NVBLK))

    tb_lin = pl.pallas_call(
        _transpose_body,
        grid=(TGRID,),
        in_specs=[pl.BlockSpec((D, VB), _qmap(q)) for q in range(4)],
        out_specs=pl.BlockSpec((VB, 4 * DP), lambda i: (i, 0)),
        out_shape=jax.ShapeDtypeStruct((VS4, 4 * DP), jnp.float32),
    )(tbT, tbT, tbT, tbT)
    pooled = _pool_kernel(x3, tb_lin.reshape(4 * VS4, DP))
    k = jnp.arange(D)
    perm = 16 * (k // 32) + (k % 32) // 2 + 32 * (k % 2)
    out = pl.pallas_call(
        _head_body,
        out_shape=jax.ShapeDtypeStruct((B, H), jnp.float32),
    )(pooled, W[:, perm], b.reshape(1, H))
    return out


# speedup vs baseline: 1.2125x; 1.0069x over previous
import functools

import jax
import jax.numpy as jnp
from jax import lax
from jax.experimental import pallas as pl
from jax.experimental.pallas import tpu as pltpu
from jax.experimental.pallas import tpu_sc as plsc

V = 1000000
B, L, D, H = 4096, 200, 64, 128
DP = D // 2
NC, NS = 2, 16
NW = NC * NS
RPW = B // NW
NCHUNK = 2
CHUNK = L // NCHUNK
BLANE = 32
G = 4
NGRP = RPW // G

VB = 6400
TGRID = 40
VS4 = TGRID * VB
NVBLK = V // VB

_mesh = plsc.VectorSubcoreMesh(
    core_axis_name="c", subcore_axis_name="s", num_cores=NC, num_subcores=NS
)


def _transpose_body(q0_ref, q1_ref, q2_ref, q3_ref, o_ref):
    for q, ref in enumerate((q0_ref, q1_ref, q2_ref, q3_ref)):
        u = lax.bitcast_convert_type(ref[...], jnp.uint32)
        ub = (u + 0x8000) >> 16
        packed_u = ub[0:DP, :] | (ub[DP:D, :] << 16)
        packed = lax.bitcast_convert_type(packed_u, jnp.float32)
        o_ref[:, pl.ds(q * DP, DP)] = lax.transpose(packed, (1, 0))


@functools.partial(
    pl.kernel,
    out_type=jax.ShapeDtypeStruct((B, D), jnp.bfloat16),
    mesh=_mesh,
    scratch_types=[
        pltpu.VMEM((RPW, NCHUNK, CHUNK), jnp.int32),
        pltpu.VMEM((2, G, NCHUNK, CHUNK, DP), jnp.float32),
        pltpu.VMEM((RPW, D), jnp.bfloat16),
        pltpu.SemaphoreType.DMA,
        pltpu.SemaphoreType.DMA,
    ],
    compiler_params=pltpu.CompilerParams(
        use_tc_tiling_on_sc=False, needs_layout_passes=False
    ),
)
def _pool_kernel(x_hbm, table_hbm, out_hbm, idx_v, rows_v, out_v, sem0, sem1):
    wid = lax.axis_index("s") * NC + lax.axis_index("c")
    base = wid * RPW
    sems = (sem0, sem1)
    pltpu.sync_copy(x_hbm.at[pl.ds(base, RPW)], idx_v)

    def start(g, p):
        for u in range(G):
            for j in range(NCHUNK):
                pltpu.async_copy(
                    table_hbm.at[idx_v.at[g * G + u, j]],
                    rows_v.at[p, u, j],
                    sems[p],
                )

    def wait(g, p):
        for u in range(G):
            for j in range(NCHUNK):
                pltpu.make_async_copy(
                    table_hbm.at[idx_v.at[g * G + u, j]],
                    rows_v.at[p, u, j],
                    sems[p],
                ).wait()

    def reduce(g, p):
        for u in range(G):
            def red_body(r, accs):
                res = list(accs)
                for j in range(NCHUNK):
                    for c in range(2):
                        v16 = rows_v[p, u, j, r, pl.ds(c * 16, 16)]
                        res[c] = jnp.maximum(
                            res[c], plsc.bitcast(v16, jnp.bfloat16)
                        )
                return tuple(res)

            init = tuple(
                jnp.full((BLANE,), -jnp.inf, jnp.bfloat16) for _ in range(2)
            )
            accs = lax.fori_loop(0, CHUNK, red_body, init)
            for c in range(2):
                out_v[g * G + u, pl.ds(c * BLANE, BLANE)] = accs[c]

    start(0, 0)
    start(1, 1)

    def grp_body(i, carry):
        for p in range(2):
            g = 2 * i + p
            wait(g, p)
            reduce(g, p)
            start(g + 2, p)
        return carry

    lax.fori_loop(0, NGRP // 2 - 1, grp_body, 0)
    for p in range(2):
        g = NGRP - 2 + p
        wait(g, p)
        reduce(g, p)
    pltpu.sync_copy(out_v, out_hbm.at[pl.ds(base, RPW)])


def _head_body(p_ref, w_ref, b_ref, o_ref):
    pooled = p_ref[...].astype(jnp.float32)
    h = lax.dot_general(
        pooled, w_ref[...], (((1,), (1,)), ((), ())),
        preferred_element_type=jnp.float32,
    )
    h = h + b_ref[...]
    s = jnp.sum(h * h, axis=1, keepdims=True)
    o_ref[...] = h * lax.rsqrt(jnp.maximum(s, 1e-24))


def kernel(x, embed_table, W, b):
    xi = x.astype(jnp.int32)
    xg = 4 * jnp.remainder(xi, VS4) + xi // VS4
    x3 = xg.reshape(B, NCHUNK, CHUNK)
    tbT = embed_table.T

    def _qmap(q):
        return lambda i: (0, jnp.minimum(q * TGRID + i, NVBLK))

    tb_lin = pl.pallas_call(
        _transpose_body,
        grid=(TGRID,),
        in_specs=[pl.BlockSpec((D, VB), _qmap(q)) for q in range(4)],
        out_specs=pl.BlockSpec((VB, 4 * DP), lambda i: (i, 0)),
        out_shape=jax.ShapeDtypeStruct((VS4, 4 * DP), jnp.float32),
    )(tbT, tbT, tbT, tbT)
    pooled = _pool_kernel(x3, tb_lin.reshape(4 * VS4, DP))
    k = jnp.arange(D)
    perm = 16 * (k // 32) + (k % 32) // 2 + 32 * (k % 2)
    out = pl.pallas_call(
        _head_body,
        out_shape=jax.ShapeDtypeStruct((B, H), jnp.float32),
    )(pooled, W[:, perm], b.reshape(1, H))
    return out
